# Initial kernel scaffold; baseline (speedup 1.0000x reference)
import jax
import jax.numpy as jnp
from jax.experimental import pallas as pl


def kernel(x, edge_index, edge_type, rel_embed, weight, rel_weight, bias):
    raise NotImplementedError("write your pallas kernel here")


# speedup vs baseline: 2.7373x; 2.7373x over previous
import functools

import jax
import jax.numpy as jnp
from jax import lax
from jax.experimental import pallas as pl
from jax.experimental.pallas import tpu as pltpu
from jax.experimental.pallas import tpu_sc as plsc

N = 10000
E = 320000
D = 128
NUM_RELS = 200

NC = 2
NS = 16
NW = NC * NS
CHUNK = 128
EPW = 10240
NCHUNK = EPW // CHUNK
IDXC = 16
NSUP = NCHUNK // IDXC
EPAD = EPW * NW
NPAD = 10240
ROWS_PT = NPAD // NS
LANES = 16


def _sc_message_accumulate(src3, dst3, et3, x, re_ext, zrows):
  mesh = plsc.VectorSubcoreMesh(core_axis_name="c", subcore_axis_name="s")

  @functools.partial(
      pl.kernel,
      out_type=jax.ShapeDtypeStruct((NC, NPAD, D), jnp.float32),
      mesh=mesh,
      scratch_types=[
          pltpu.VMEM((IDXC, CHUNK), jnp.int32),
          pltpu.VMEM((IDXC, CHUNK), jnp.int32),
          pltpu.VMEM((IDXC, CHUNK), jnp.int32),
          pltpu.VMEM((CHUNK, D), jnp.float32),
          pltpu.VMEM((CHUNK, D), jnp.float32),
          pltpu.VMEM_SHARED((NPAD, D), jnp.float32),
          pltpu.SemaphoreType.DMA,
          pltpu.SemaphoreType.DMA,
      ],
  )
  def k(src_hbm, dst_hbm, et_hbm, x_hbm, re_hbm, z_hbm, out_hbm,
        src_v, dst_v, et_v, xr_v, rr_v, acc_sh, sem1, sem2):
    cid = lax.axis_index("c")
    sid = lax.axis_index("s")
    wid = cid * NS + sid

    pltpu.sync_copy(z_hbm, acc_sh.at[pl.ds(sid * ROWS_PT, ROWS_PT)])
    plsc.subcore_barrier()

    def super_body(sc, carry):
      pltpu.sync_copy(src_hbm.at[wid, sc], src_v)
      pltpu.sync_copy(dst_hbm.at[wid, sc], dst_v)
      pltpu.sync_copy(et_hbm.at[wid, sc], et_v)

      def chunk_body(c, carry1):
        pltpu.async_copy(x_hbm.at[src_v.at[c]], xr_v, sem1).wait()
        pltpu.async_copy(re_hbm.at[et_v.at[c]], rr_v, sem2).wait()

        def mul_body(e, carry2):
          for j in range(D // LANES):
            s = pl.ds(j * LANES, LANES)
            xr_v[e, s] = xr_v[e, s] * rr_v[e, s]
          return carry2

        lax.fori_loop(0, CHUNK, mul_body, 0)
        pltpu.sync_copy(xr_v, acc_sh.at[dst_v.at[c]], add=True)
        return carry1

      lax.fori_loop(0, IDXC, chunk_body, 0)
      return carry

    lax.fori_loop(0, NSUP, super_body, 0)
    plsc.subcore_barrier()

    r0 = sid * ROWS_PT
    pltpu.sync_copy(acc_sh.at[pl.ds(r0, ROWS_PT)],
                    out_hbm.at[cid, pl.ds(r0, ROWS_PT)])

  return k(src3, dst3, et3, x, re_ext, zrows)


def _tc_out_matmul(partials, weight, bias2d):
  BM = 1024

  def body(p_ref, w_ref, b_ref, o_ref):
    acc = jnp.dot(p_ref[0] + p_ref[1], w_ref[...],
                  preferred_element_type=jnp.float32)
    o_ref[...] = acc + b_ref[...]

  return pl.pallas_call(
      body,
      grid=(NPAD // BM,),
      in_specs=[
          pl.BlockSpec((NC, BM, D), lambda i: (0, i, 0)),
          pl.BlockSpec((D, D), lambda i: (0, 0)),
          pl.BlockSpec((1, D), lambda i: (0, 0)),
      ],
      out_specs=pl.BlockSpec((BM, D), lambda i: (i, 0)),
      out_shape=jax.ShapeDtypeStruct((NPAD, D), jnp.float32),
  )(partials, weight, bias2d)


def _tc_rel_matmul(re_pad, rel_weight):
  def body(r_ref, w_ref, o_ref):
    o_ref[...] = jnp.dot(r_ref[...], w_ref[...],
                         preferred_element_type=jnp.float32)

  return pl.pallas_call(
      body,
      out_shape=jax.ShapeDtypeStruct((re_pad.shape[0], D), jnp.float32),
  )(re_pad, rel_weight)


def kernel(x, edge_index, edge_type, rel_embed, weight, rel_weight, bias):
  src = edge_index[0]
  dst = edge_index[1]
  npad = EPAD - E
  src3 = jnp.concatenate(
      [src, jnp.zeros((npad,), jnp.int32)]).reshape(NW, NSUP, IDXC, CHUNK)
  dst3 = jnp.concatenate(
      [dst, jnp.zeros((npad,), jnp.int32)]).reshape(NW, NSUP, IDXC, CHUNK)
  et3 = jnp.concatenate(
      [edge_type, jnp.full((npad,), 2 * NUM_RELS, jnp.int32)]
  ).reshape(NW, NSUP, IDXC, CHUNK)

  re_ext = jnp.concatenate(
      [rel_embed, jnp.zeros((1, D), rel_embed.dtype)], axis=0)
  zrows = jnp.zeros((ROWS_PT, D), jnp.float32)

  partials = _sc_message_accumulate(src3, dst3, et3, x, re_ext, zrows)
  out = _tc_out_matmul(partials, weight, bias.reshape(1, D))[:N]

  re_pad = jnp.concatenate(
      [re_ext, jnp.zeros((7, D), rel_embed.dtype)], axis=0)
  rel_out = _tc_rel_matmul(re_pad, rel_weight)[:2 * NUM_RELS + 1]
  return (out, rel_out)


# speedup vs baseline: 5.3262x; 1.9458x over previous
import functools

import jax
import jax.numpy as jnp
from jax import lax
from jax.experimental import pallas as pl
from jax.experimental.pallas import tpu as pltpu
from jax.experimental.pallas import tpu_sc as plsc

N = 10000
E = 320000
D = 128
NUM_RELS = 200

NC = 2
NS = 16
NW = NC * NS
CHUNK = 80
EPW = 10240
NCHUNK = EPW // CHUNK
IDXC = 16
NSUP = NCHUNK // IDXC
NPAIR = IDXC // 2
EPAD = EPW * NW
NPAD = 10112
ROWS_PT = NPAD // NS
LANES = 16


def _sc_message_accumulate(src3, dst3, et3, x, re_ext, zrows):
  mesh = plsc.VectorSubcoreMesh(core_axis_name="c", subcore_axis_name="s")

  @functools.partial(
      pl.kernel,
      out_type=jax.ShapeDtypeStruct((NC, NPAD, D), jnp.float32),
      mesh=mesh,
      scratch_types=[
          pltpu.VMEM((IDXC, CHUNK), jnp.int32),
          pltpu.VMEM((IDXC, CHUNK), jnp.int32),
          pltpu.VMEM((IDXC, CHUNK), jnp.int32),
          pltpu.VMEM((CHUNK, D), jnp.float32),
          pltpu.VMEM((CHUNK, D), jnp.float32),
          pltpu.VMEM((CHUNK, D), jnp.float32),
          pltpu.VMEM((CHUNK, D), jnp.float32),
          pltpu.VMEM_SHARED((NPAD, D), jnp.float32),
          pltpu.SemaphoreType.DMA,
          pltpu.SemaphoreType.DMA,
          pltpu.SemaphoreType.DMA,
          pltpu.SemaphoreType.DMA,
      ],
  )
  def k(src_hbm, dst_hbm, et_hbm, x_hbm, re_hbm, z_hbm, out_hbm,
        src_v, dst_v, et_v, xr_a, rr_a, xr_b, rr_b, acc_sh,
        sem_ga, sem_gb, sem_sa, sem_sb):
    cid = lax.axis_index("c")
    sid = lax.axis_index("s")
    wid = cid * NS + sid

    pltpu.sync_copy(z_hbm, acc_sh.at[pl.ds(sid * ROWS_PT, ROWS_PT)])
    plsc.subcore_barrier()

    def issue_gather(c, xr, rr, sem):
      pltpu.async_copy(x_hbm.at[src_v.at[c]], xr, sem)
      pltpu.async_copy(re_hbm.at[et_v.at[c]], rr, sem)

    def wait_gather(c, xr, rr, sem):
      pltpu.make_async_copy(x_hbm.at[src_v.at[c]], xr, sem).wait()
      pltpu.make_async_copy(re_hbm.at[et_v.at[c]], rr, sem).wait()

    def mul(xr, rr):
      def mul_body(e, carry2):
        for j in range(D // LANES):
          s = pl.ds(j * LANES, LANES)
          xr[e, s] = xr[e, s] * rr[e, s]
        return carry2

      lax.fori_loop(0, CHUNK, mul_body, 0)

    def super_body(sc, carry):
      pltpu.sync_copy(src_hbm.at[wid, sc], src_v)
      pltpu.sync_copy(dst_hbm.at[wid, sc], dst_v)
      pltpu.sync_copy(et_hbm.at[wid, sc], et_v)

      issue_gather(0, xr_a, rr_a, sem_ga)
      issue_gather(1, xr_b, rr_b, sem_gb)

      def pair_body(p, carry1):
        c0 = 2 * p
        c1 = c0 + 1
        wait_gather(c0, xr_a, rr_a, sem_ga)
        mul(xr_a, rr_a)
        scat_a = pltpu.async_copy(xr_a, acc_sh.at[dst_v.at[c0]], sem_sa,
                                  add=True)
        wait_gather(c1, xr_b, rr_b, sem_gb)
        mul(xr_b, rr_b)
        scat_b = pltpu.async_copy(xr_b, acc_sh.at[dst_v.at[c1]], sem_sb,
                                  add=True)

        @pl.when(p + 1 < NPAIR)
        def _():
          scat_a.wait()
          issue_gather(c0 + 2, xr_a, rr_a, sem_ga)
          scat_b.wait()
          issue_gather(c1 + 2, xr_b, rr_b, sem_gb)

        return carry1

      lax.fori_loop(0, NPAIR, pair_body, 0)
      last0 = IDXC - 2
      last1 = IDXC - 1
      pltpu.make_async_copy(xr_a, acc_sh.at[dst_v.at[last0]], sem_sa).wait()
      pltpu.make_async_copy(xr_b, acc_sh.at[dst_v.at[last1]], sem_sb).wait()
      return carry

    lax.fori_loop(0, NSUP, super_body, 0)
    plsc.subcore_barrier()

    r0 = sid * ROWS_PT
    pltpu.sync_copy(acc_sh.at[pl.ds(r0, ROWS_PT)],
                    out_hbm.at[cid, pl.ds(r0, ROWS_PT)])

  return k(src3, dst3, et3, x, re_ext, zrows)


def _tc_out_matmul(partials, weight, bias2d):
  BM = 1264

  def body(p_ref, w_ref, b_ref, o_ref):
    acc = jnp.dot(p_ref[0] + p_ref[1], w_ref[...],
                  preferred_element_type=jnp.float32)
    o_ref[...] = acc + b_ref[...]

  return pl.pallas_call(
      body,
      grid=(NPAD // BM,),
      in_specs=[
          pl.BlockSpec((NC, BM, D), lambda i: (0, i, 0)),
          pl.BlockSpec((D, D), lambda i: (0, 0)),
          pl.BlockSpec((1, D), lambda i: (0, 0)),
      ],
      out_specs=pl.BlockSpec((BM, D), lambda i: (i, 0)),
      out_shape=jax.ShapeDtypeStruct((NPAD, D), jnp.float32),
  )(partials, weight, bias2d)


def _tc_rel_matmul(re_pad, rel_weight):
  def body(r_ref, w_ref, o_ref):
    o_ref[...] = jnp.dot(r_ref[...], w_ref[...],
                         preferred_element_type=jnp.float32)

  return pl.pallas_call(
      body,
      out_shape=jax.ShapeDtypeStruct((re_pad.shape[0], D), jnp.float32),
  )(re_pad, rel_weight)


def kernel(x, edge_index, edge_type, rel_embed, weight, rel_weight, bias):
  src = edge_index[0]
  dst = edge_index[1]
  npad = EPAD - E
  src3 = jnp.concatenate(
      [src, jnp.zeros((npad,), jnp.int32)]).reshape(NW, NSUP, IDXC, CHUNK)
  dst3 = jnp.concatenate(
      [dst, jnp.zeros((npad,), jnp.int32)]).reshape(NW, NSUP, IDXC, CHUNK)
  et3 = jnp.concatenate(
      [edge_type, jnp.full((npad,), 2 * NUM_RELS, jnp.int32)]
  ).reshape(NW, NSUP, IDXC, CHUNK)

  re_ext = jnp.concatenate(
      [rel_embed, jnp.zeros((1, D), rel_embed.dtype)], axis=0)
  zrows = jnp.zeros((ROWS_PT, D), jnp.float32)

  partials = _sc_message_accumulate(src3, dst3, et3, x, re_ext, zrows)
  out = _tc_out_matmul(partials, weight, bias.reshape(1, D))[:N]

  re_pad = jnp.concatenate(
      [re_ext, jnp.zeros((7, D), rel_embed.dtype)], axis=0)
  rel_out = _tc_rel_matmul(re_pad, rel_weight)[:2 * NUM_RELS + 1]
  return (out, rel_out)
